# baseline (speedup 1.0000x reference)
import functools

import jax
import jax.numpy as jnp
from jax import lax
from jax.experimental import pallas as pl
from jax.experimental.pallas import tpu as pltpu
from jax.experimental.pallas import tpu_sc as plsc

FILTER_THRESHOLD = 0.95
FILTER_FACTOR = 0.5

_SC_ROWS = 1280
_CHUNK_ROWS = 8
_UNROLL = 8
_NBUF = 3
_TC_BLOCK_ROWS = 256

_BCAST_DNUMS = lax.GatherDimensionNumbers(
    offset_dims=(), collapsed_slice_dims=(0,), start_index_map=(0,)
)


def _lane_broadcast(vec, idx, lanes):
    return lax.gather(
        vec,
        jnp.full((lanes, 1), idx, jnp.int32),
        _BCAST_DNUMS,
        (1,),
        mode=lax.GatherScatterMode.PROMISE_IN_BOUNDS,
    )


def _make_sc_kernel(B, R):
    info = plsc.get_sparse_core_info()
    nw = info.num_cores * info.num_subcores
    lanes = info.num_lanes
    rows_w = R // nw
    n_chunks = rows_w // _CHUNK_ROWS
    mesh = plsc.VectorSubcoreMesh(core_axis_name="c", subcore_axis_name="s")

    @functools.partial(
        pl.kernel,
        mesh=mesh,
        compiler_params=pltpu.CompilerParams(use_tc_tiling_on_sc=True),
        out_type=jax.ShapeDtypeStruct((R, B), jnp.float32),
        scratch_types=[
            pltpu.VMEM((_NBUF, _CHUNK_ROWS, B), jnp.float32),
            pltpu.SemaphoreType.DMA((_NBUF,)),
            pltpu.SemaphoreType.DMA((_NBUF,)),
        ],
    )
    def sc_filter(scores_hbm, out_hbm, bufs, sem_in, sem_out):
        wid = lax.axis_index("s") * info.num_cores + lax.axis_index("c")
        base = wid * rows_w
        lane = lax.iota(jnp.int32, lanes)

        def start_in(g):
            row0 = base + g * _CHUNK_ROWS
            pltpu.async_copy(
                scores_hbm.at[pl.ds(row0, _CHUNK_ROWS)],
                bufs.at[g % _NBUF],
                sem_in.at[g % _NBUF],
            )

        def start_out(g):
            row0 = base + g * _CHUNK_ROWS
            pltpu.async_copy(
                bufs.at[g % _NBUF],
                out_hbm.at[pl.ds(row0, _CHUNK_ROWS)],
                sem_out.at[g % _NBUF],
            )

        def wait_out(g):
            row0 = base + g * _CHUNK_ROWS
            pltpu.make_async_copy(
                bufs.at[g % _NBUF],
                out_hbm.at[pl.ds(row0, _CHUNK_ROWS)],
                sem_out.at[g % _NBUF],
            ).wait()

        def wait_in(g):
            row0 = base + g * _CHUNK_ROWS
            pltpu.make_async_copy(
                scores_hbm.at[pl.ds(row0, _CHUNK_ROWS)],
                bufs.at[g % _NBUF],
                sem_in.at[g % _NBUF],
            ).wait()

        start_in(0)

        def chunk_body(g, carry):
            @pl.when(g >= _NBUF - 1)
            def _():
                wait_out(g - (_NBUF - 1))

            @pl.when(g + 1 < n_chunks)
            def _():
                start_in(g + 1)

            wait_in(g)
            row0 = base + g * _CHUNK_ROWS
            b = g % _NBUF
            calign = (row0 // lanes) * lanes

            for r in range(_CHUNK_ROWS):
                dlane = row0 - calign + r
                dvec = bufs[b, r, pl.ds(calign, lanes)]
                th = _lane_broadcast(dvec, dlane, lanes) * FILTER_THRESHOLD

                @plsc.parallel_loop(0, calign, step=lanes, unroll=_UNROLL)
                def left_body(c, b=b, r=r, th=th):
                    sl = pl.ds(c, lanes)
                    v = bufs[b, r, sl]
                    bufs[b, r, sl] = jnp.where(v > th, v * FILTER_FACTOR, v)

                @plsc.parallel_loop(calign + lanes, B, step=lanes, unroll=_UNROLL)
                def right_body(c, b=b, r=r, th=th):
                    sl = pl.ds(c, lanes)
                    v = bufs[b, r, sl]
                    bufs[b, r, sl] = jnp.where(v > th, v * FILTER_FACTOR, v)

                m = (dvec > th) & (lane != dlane)
                bufs[b, r, pl.ds(calign, lanes)] = jnp.where(
                    m, dvec * FILTER_FACTOR, dvec
                )

            start_out(g)
            return carry

        lax.fori_loop(0, n_chunks, chunk_body, 0)
        for t in range(_NBUF - 1):
            wait_out(n_chunks - (_NBUF - 1) + t)

    return sc_filter


def _tc_filter_block(scores_ref, out_ref, *, block_off):
    i = pl.program_id(0) + block_off
    blk = scores_ref[...]
    rows = blk.shape[0]
    sub = scores_ref[:, pl.ds(i * rows, rows)]
    r_iota = jax.lax.broadcasted_iota(jnp.int32, (rows, rows), 0)
    c_iota = jax.lax.broadcasted_iota(jnp.int32, (rows, rows), 1)
    eq = r_iota == c_iota
    diag = jnp.max(jnp.where(eq, sub, -jnp.inf), axis=1, keepdims=True)
    thresh = FILTER_THRESHOLD * diag
    out_ref[...] = jnp.where(blk > thresh, blk * FILTER_FACTOR, blk)
    sub_filtered = jnp.where(sub > thresh, sub * FILTER_FACTOR, sub)
    out_ref[:, pl.ds(i * rows, rows)] = jnp.where(eq, sub, sub_filtered)


def _tc_call(scores, r0):
    B = scores.shape[0]
    rows = _TC_BLOCK_ROWS
    block_off = r0 // rows
    grid = ((B - r0) // rows,)
    return pl.pallas_call(
        functools.partial(_tc_filter_block, block_off=block_off),
        grid=grid,
        in_specs=[pl.BlockSpec((rows, B), lambda i: (i + block_off, 0))],
        out_specs=pl.BlockSpec((rows, B), lambda i: (i, 0)),
        out_shape=jax.ShapeDtypeStruct((B - r0, B), scores.dtype),
    )(scores)


def kernel(scores):
    B = scores.shape[0]
    sc_out = _make_sc_kernel(B, _SC_ROWS)(scores)
    tc_out = _tc_call(scores, _SC_ROWS)
    return jnp.concatenate([sc_out, tc_out], axis=0)


# speedup vs baseline: 1.2102x; 1.2102x over previous
import functools

import jax
import jax.numpy as jnp
from jax import lax
from jax.experimental import pallas as pl
from jax.experimental.pallas import tpu as pltpu
from jax.experimental.pallas import tpu_sc as plsc

FILTER_THRESHOLD = 0.95
FILTER_FACTOR = 0.5

_CHUNK_ROWS = 4
_UNROLL = 8
_NBUF = 6
_PREF = 2

_BCAST_DNUMS = lax.GatherDimensionNumbers(
    offset_dims=(), collapsed_slice_dims=(0,), start_index_map=(0,)
)


def _lane_broadcast(vec, idx, lanes):
    return lax.gather(
        vec,
        jnp.full((lanes, 1), idx, jnp.int32),
        _BCAST_DNUMS,
        (1,),
        mode=lax.GatherScatterMode.PROMISE_IN_BOUNDS,
    )


def _make_sc_kernel(B):
    info = plsc.get_sparse_core_info()
    nw = info.num_cores * info.num_subcores
    lanes = info.num_lanes
    rows_w = B // nw
    n_chunks = rows_w // _CHUNK_ROWS
    mesh = plsc.VectorSubcoreMesh(core_axis_name="c", subcore_axis_name="s")

    @functools.partial(
        pl.kernel,
        mesh=mesh,
        compiler_params=pltpu.CompilerParams(use_tc_tiling_on_sc=True),
        out_type=jax.ShapeDtypeStruct((B, B), jnp.float32),
        scratch_types=[
            pltpu.VMEM((_NBUF, _CHUNK_ROWS, B), jnp.float32),
            pltpu.SemaphoreType.DMA((_NBUF,)),
            pltpu.SemaphoreType.DMA((_NBUF,)),
        ],
    )
    def sc_filter(scores_hbm, out_hbm, bufs, sem_in, sem_out):
        wid = lax.axis_index("s") * info.num_cores + lax.axis_index("c")
        base = wid * rows_w
        lane = lax.iota(jnp.int32, lanes)

        def start_in(g):
            row0 = base + g * _CHUNK_ROWS
            pltpu.async_copy(
                scores_hbm.at[pl.ds(row0, _CHUNK_ROWS)],
                bufs.at[g % _NBUF],
                sem_in.at[g % _NBUF],
            )

        def start_out(g):
            row0 = base + g * _CHUNK_ROWS
            pltpu.async_copy(
                bufs.at[g % _NBUF],
                out_hbm.at[pl.ds(row0, _CHUNK_ROWS)],
                sem_out.at[g % _NBUF],
            )

        def wait_out(g):
            row0 = base + g * _CHUNK_ROWS
            pltpu.make_async_copy(
                bufs.at[g % _NBUF],
                out_hbm.at[pl.ds(row0, _CHUNK_ROWS)],
                sem_out.at[g % _NBUF],
            ).wait()

        def wait_in(g):
            row0 = base + g * _CHUNK_ROWS
            pltpu.make_async_copy(
                scores_hbm.at[pl.ds(row0, _CHUNK_ROWS)],
                bufs.at[g % _NBUF],
                sem_in.at[g % _NBUF],
            ).wait()

        for t in range(_PREF):
            start_in(t)

        def chunk_body(g, carry):
            @pl.when(g >= _NBUF - _PREF)
            def _():
                wait_out(g - (_NBUF - _PREF))

            @pl.when(g + _PREF < n_chunks)
            def _():
                start_in(g + _PREF)

            wait_in(g)
            row0 = base + g * _CHUNK_ROWS
            b = g % _NBUF
            calign = (row0 // lanes) * lanes

            for r in range(_CHUNK_ROWS):
                dlane = row0 - calign + r
                dvec = bufs[b, r, pl.ds(calign, lanes)]
                th = _lane_broadcast(dvec, dlane, lanes) * FILTER_THRESHOLD

                @plsc.parallel_loop(0, calign, step=lanes, unroll=_UNROLL)
                def left_body(c, b=b, r=r, th=th):
                    sl = pl.ds(c, lanes)
                    v = bufs[b, r, sl]
                    bufs[b, r, sl] = jnp.where(v > th, v * FILTER_FACTOR, v)

                @plsc.parallel_loop(calign + lanes, B, step=lanes, unroll=_UNROLL)
                def right_body(c, b=b, r=r, th=th):
                    sl = pl.ds(c, lanes)
                    v = bufs[b, r, sl]
                    bufs[b, r, sl] = jnp.where(v > th, v * FILTER_FACTOR, v)

                m = (dvec > th) & (lane != dlane)
                bufs[b, r, pl.ds(calign, lanes)] = jnp.where(
                    m, dvec * FILTER_FACTOR, dvec
                )

            start_out(g)
            return carry

        lax.fori_loop(0, n_chunks, chunk_body, 0)
        for t in range(_NBUF - _PREF):
            wait_out(n_chunks - (_NBUF - _PREF) + t)

    return sc_filter


def kernel(scores):
    B = scores.shape[0]
    return _make_sc_kernel(B)(scores)


# speedup vs baseline: 1.4418x; 1.1914x over previous
import functools

import jax
import jax.numpy as jnp
from jax import lax
from jax.experimental import pallas as pl
from jax.experimental.pallas import tpu as pltpu
from jax.experimental.pallas import tpu_sc as plsc

FILTER_THRESHOLD = 0.95
FILTER_FACTOR = 0.5

_CHUNK_ROWS = 8
_UNROLL = 8
_NBUF = 3

_BCAST_DNUMS = lax.GatherDimensionNumbers(
    offset_dims=(), collapsed_slice_dims=(0,), start_index_map=(0,)
)


def _lane_broadcast(vec, idx, lanes):
    return lax.gather(
        vec,
        jnp.full((lanes, 1), idx, jnp.int32),
        _BCAST_DNUMS,
        (1,),
        mode=lax.GatherScatterMode.PROMISE_IN_BOUNDS,
    )


def _make_sc_kernel(B):
    info = plsc.get_sparse_core_info()
    nw = info.num_cores * info.num_subcores
    lanes = info.num_lanes
    rows_w = B // nw
    n_chunks = rows_w // _CHUNK_ROWS
    mesh = plsc.VectorSubcoreMesh(core_axis_name="c", subcore_axis_name="s")

    @functools.partial(
        pl.kernel,
        mesh=mesh,
        compiler_params=pltpu.CompilerParams(use_tc_tiling_on_sc=True),
        out_type=jax.ShapeDtypeStruct((B, B), jnp.float32),
        scratch_types=[
            pltpu.VMEM((_NBUF, _CHUNK_ROWS, B), jnp.float32),
            pltpu.SemaphoreType.DMA((_NBUF,)),
            pltpu.SemaphoreType.DMA((_NBUF,)),
        ],
    )
    def sc_filter(scores_hbm, out_hbm, bufs, sem_in, sem_out):
        wid = lax.axis_index("s") * info.num_cores + lax.axis_index("c")
        base = wid * rows_w
        lane = lax.iota(jnp.int32, lanes)

        def start_in(g):
            row0 = base + g * _CHUNK_ROWS
            pltpu.async_copy(
                scores_hbm.at[pl.ds(row0, _CHUNK_ROWS)],
                bufs.at[g % _NBUF],
                sem_in.at[g % _NBUF],
            )

        def start_out(g):
            row0 = base + g * _CHUNK_ROWS
            pltpu.async_copy(
                bufs.at[g % _NBUF],
                out_hbm.at[pl.ds(row0, _CHUNK_ROWS)],
                sem_out.at[g % _NBUF],
            )

        def wait_out(g):
            row0 = base + g * _CHUNK_ROWS
            pltpu.make_async_copy(
                bufs.at[g % _NBUF],
                out_hbm.at[pl.ds(row0, _CHUNK_ROWS)],
                sem_out.at[g % _NBUF],
            ).wait()

        def wait_in(g):
            row0 = base + g * _CHUNK_ROWS
            pltpu.make_async_copy(
                scores_hbm.at[pl.ds(row0, _CHUNK_ROWS)],
                bufs.at[g % _NBUF],
                sem_in.at[g % _NBUF],
            ).wait()

        start_in(0)

        def chunk_body(g, carry):
            @pl.when(g >= _NBUF - 1)
            def _():
                wait_out(g - (_NBUF - 1))

            @pl.when(g + 1 < n_chunks)
            def _():
                start_in(g + 1)

            wait_in(g)
            row0 = base + g * _CHUNK_ROWS
            b = g % _NBUF
            calign = (row0 // lanes) * lanes

            for r in range(_CHUNK_ROWS):
                dlane = row0 - calign + r
                dvec = bufs[b, r, pl.ds(calign, lanes)]
                th = _lane_broadcast(dvec, dlane, lanes) * FILTER_THRESHOLD

                @plsc.parallel_loop(0, calign, step=lanes, unroll=_UNROLL)
                def left_body(c, b=b, r=r, th=th):
                    sl = pl.ds(c, lanes)
                    v = bufs[b, r, sl]
                    bufs[b, r, sl] = jnp.where(v > th, v * FILTER_FACTOR, v)

                @plsc.parallel_loop(calign + lanes, B, step=lanes, unroll=_UNROLL)
                def right_body(c, b=b, r=r, th=th):
                    sl = pl.ds(c, lanes)
                    v = bufs[b, r, sl]
                    bufs[b, r, sl] = jnp.where(v > th, v * FILTER_FACTOR, v)

                m = (dvec > th) & (lane != dlane)
                bufs[b, r, pl.ds(calign, lanes)] = jnp.where(
                    m, dvec * FILTER_FACTOR, dvec
                )

            start_out(g)
            return carry

        lax.fori_loop(0, n_chunks, chunk_body, 0)
        for t in range(_NBUF - 1):
            wait_out(n_chunks - (_NBUF - 1) + t)

    return sc_filter


def kernel(scores):
    B = scores.shape[0]
    return _make_sc_kernel(B)(scores)
